# Initial kernel scaffold; baseline (speedup 1.0000x reference)
import jax
import jax.numpy as jnp
from jax.experimental import pallas as pl


def kernel(x, w_b1, b_b1, w_b2a, b_b2a, w_b2b, b_b2b, w_b3a, b_b3a, w_b3b, b_b3b, w_b4, b_b4):
    raise NotImplementedError("write your pallas kernel here")


# speedup vs baseline: 5.9892x; 5.9892x over previous
import functools

import jax
import jax.numpy as jnp
from jax import lax
from jax.experimental import pallas as pl
from jax.experimental.pallas import tpu as pltpu


def _pack(w):
    co, ci, k, _ = w.shape
    return jnp.transpose(w, (0, 2, 3, 1)).reshape(co, k * k * ci)


def _fused_kernel(h, w, hw, hwp, k3, k5, c1, c3, c5, cr3,
                  x_ref, wred_ref, w1_ref, w3_ref, w5_ref, wp_ref,
                  bred_ref, b1_ref, b3_ref, b5_ref, bp_ref, o_ref):
    f32 = jnp.float32
    bf16 = jnp.bfloat16
    xf = x_ref[0]
    xb = xf.astype(bf16)

    pos = lax.broadcasted_iota(jnp.int32, (1, hwp), 1)
    yy = pos // w
    xx = pos - yy * w
    in_img = pos < hw

    def shift(a, s):
        if s == 0:
            return a
        k = s % hwp
        return jnp.concatenate([a[:, k:], a[:, :k]], axis=-1)

    def bounds_mask(oy, ox):
        m = in_img
        if oy:
            m = m & (yy + oy >= 0) & (yy + oy < h)
        if ox:
            m = m & (xx + ox >= 0) & (xx + ox < w)
        return m

    def conv(w_r, t, b_r):
        y = jnp.dot(w_r[...], t, preferred_element_type=f32)
        return jnp.maximum(y + b_r[...], 0.0)

    yred = conv(wred_ref, xb, bred_ref)
    r3x3 = yred[:cr3]
    r5x5 = yred[cr3:]

    def taps(r, k):
        p = (k - 1) // 2
        cols = []
        for oy in range(-p, p + 1):
            for ox in range(-p, p + 1):
                mf = jnp.where(bounds_mask(oy, ox), f32(1.0), f32(0.0))
                cols.append((shift(r, oy * w + ox) * mf).astype(bf16))
        return jnp.concatenate(cols, axis=0)

    y3 = conv(w3_ref, taps(r3x3, k3), b3_ref)
    y5 = conv(w5_ref, taps(r5x5, k5), b5_ref)
    y1 = conv(w1_ref, xb, b1_ref)

    neg = f32(-1e30)
    hmax = xf
    for ox in (-1, 1):
        m = (xx + ox >= 0) & (xx + ox < w)
        hmax = jnp.maximum(hmax, jnp.where(m, shift(xf, ox), neg))
    pooled = hmax
    for oy in (-1, 1):
        m = (yy + oy >= 0) & (yy + oy < h)
        pooled = jnp.maximum(pooled, jnp.where(m, shift(hmax, oy * w), neg))
    y4 = conv(wp_ref, pooled.astype(bf16), bp_ref)

    o_ref[0, 0:c1] = y1
    o_ref[0, c1:c1 + c3] = y3
    o_ref[0, c1 + c3:c1 + c3 + c5] = y5
    o_ref[0, c1 + c3 + c5:] = y4


@jax.jit
def kernel(x, w_b1, b_b1, w_b2a, b_b2a, w_b2b, b_b2b,
           w_b3a, b_b3a, w_b3b, b_b3b, w_b4, b_b4):
    n, cin, h, w = x.shape
    hw = h * w
    hwp = (hw + 127) // 128 * 128
    k3, k5 = w_b2b.shape[2], w_b3b.shape[2]
    c1, c3, c5, cp = w_b1.shape[0], w_b2b.shape[0], w_b3b.shape[0], w_b4.shape[0]
    cr3, cr5 = w_b2a.shape[0], w_b3a.shape[0]
    ct = c1 + c3 + c5 + cp

    xr = x.astype(jnp.float32).reshape(n, cin, hw)
    x_ncm = jnp.pad(xr, ((0, 0), (0, 0), (0, hwp - hw)))

    bf16 = jnp.bfloat16
    wred = jnp.concatenate([_pack(w_b2a), _pack(w_b3a)], axis=0).astype(bf16)
    w1p = _pack(w_b1).astype(bf16)
    w3p = _pack(w_b2b).astype(bf16)
    w5p = _pack(w_b3b).astype(bf16)
    wpp = _pack(w_b4).astype(bf16)
    bred = jnp.concatenate([b_b2a, b_b3a]).reshape(-1, 1)
    b1r = b_b1.reshape(-1, 1)
    b3r = b_b2b.reshape(-1, 1)
    b5r = b_b3b.reshape(-1, 1)
    bpr = b_b4.reshape(-1, 1)

    kern = functools.partial(_fused_kernel, h, w, hw, hwp, k3, k5,
                             c1, c3, c5, cr3)
    out = pl.pallas_call(
        kern,
        out_shape=jax.ShapeDtypeStruct((n, ct, hwp), jnp.float32),
        grid=(n,),
        in_specs=[
            pl.BlockSpec((1, cin, hwp), lambda i: (i, 0, 0)),
            pl.BlockSpec((cr3 + cr5, cin), lambda i: (0, 0)),
            pl.BlockSpec((c1, cin), lambda i: (0, 0)),
            pl.BlockSpec((c3, k3 * k3 * cr3), lambda i: (0, 0)),
            pl.BlockSpec((c5, k5 * k5 * cr5), lambda i: (0, 0)),
            pl.BlockSpec((cp, cin), lambda i: (0, 0)),
            pl.BlockSpec((cr3 + cr5, 1), lambda i: (0, 0)),
            pl.BlockSpec((c1, 1), lambda i: (0, 0)),
            pl.BlockSpec((c3, 1), lambda i: (0, 0)),
            pl.BlockSpec((c5, 1), lambda i: (0, 0)),
            pl.BlockSpec((cp, 1), lambda i: (0, 0)),
        ],
        out_specs=pl.BlockSpec((1, ct, hwp), lambda i: (i, 0, 0)),
        compiler_params=pltpu.CompilerParams(
            dimension_semantics=("parallel",),
            vmem_limit_bytes=32 * 1024 * 1024),
    )(x_ncm, wred, w1p, w3p, w5p, wpp, bred, b1r, b3r, b5r, bpr)
    return out[:, :, :hw].reshape(n, ct, h, w)
